# baseline (speedup 1.0000x reference)
import numpy as np
import jax
import jax.numpy as jnp
from jax import lax
from jax.experimental import pallas as pl
from jax.experimental.pallas import tpu as pltpu

_TOPN = 100
_B = 4
_BIGN = np.int32(2**30)

_ANCHORS = np.array(
    [[-38., -16., 0., 53., 31., 15.],
     [-84., -40., 0., 99., 55., 15.],
     [-176., -88., 0., 191., 103., 15.],
     [-360., -184., 0., 375., 199., 15.],
     [-24., -24., 0., 39., 39., 15.],
     [-56., -56., 0., 71., 71., 15.],
     [-120., -120., 0., 135., 135., 15.],
     [-248., -248., 0., 263., 263., 15.],
     [-14., -36., 0., 29., 51., 15.],
     [-36., -80., 0., 51., 95., 15.],
     [-80., -168., 0., 95., 183., 15.],
     [-168., -344., 0., 183., 359., 15.]],
    dtype=np.float32)


def _proposal_kernel(scores_ref, bbox_ref, im_ref, out_ref, *scratches):
    for b in range(_B):
        scratches[b][...] = scores_ref[b]

    q3 = lax.broadcasted_iota(jnp.int32, (6, 128, 128), 0)
    j3 = lax.broadcasted_iota(jnp.int32, (6, 128, 128), 1)
    c3 = lax.broadcasted_iota(jnp.int32, (6, 128, 128), 2)
    m3 = (q3 * 128 + j3) * 128 + c3
    a3 = m3 // 8192
    n3 = (m3 - a3 * 8192) * 12 + a3

    lane = lax.broadcasted_iota(jnp.int32, (1, 128), 1)
    qi = lax.broadcasted_iota(jnp.int32, (6, 128), 0)
    ji = lax.broadcasted_iota(jnp.int32, (6, 128), 1)
    cio = lax.broadcasted_iota(jnp.int32, (1, 1, 128), 2)

    R0, Rn0 = [], []
    for b in range(_B):
        S3 = scores_ref[b]
        Rb = jnp.max(S3, axis=2)
        R0.append(Rb)
        Rn0.append(jnp.min(jnp.where(S3 == Rb[:, :, None], n3, _BIGN),
                           axis=2))

    def body(i, carry):
        R, Rn, selv, seln = [list(x) for x in carry]
        for b in range(_B):
            v = jnp.max(R[b], keepdims=True).reshape(1, 1)
            nsel_a = jnp.min(jnp.where(R[b] == v, Rn[b], _BIGN),
                             keepdims=True).reshape(1, 1)
            selv[b] = jnp.where(lane == i, v, selv[b])
            seln[b] = jnp.where(lane == i, nsel_a, seln[b])
            nsel = nsel_a[0, 0]
            a = nsel % 12
            p = nsel // 12
            m = a * 8192 + p
            q = m // 16384
            j = (m // 128) % 128
            row = scratches[b][pl.ds(q, 1), pl.ds(j, 1), :]
            mrow = (q * 128 + j) * 128 + cio
            arow = mrow // 8192
            nrow = (mrow - arow * 8192) * 12 + arow
            row = jnp.where(nrow == nsel_a[:, :, None], -jnp.inf, row)
            scratches[b][pl.ds(q, 1), pl.ds(j, 1), :] = row
            vr = jnp.max(row, keepdims=True).reshape(1, 1)
            nr = jnp.min(jnp.where(row == vr[:, :, None], nrow, _BIGN),
                         keepdims=True).reshape(1, 1)
            hit = (qi == q) & (ji == j)
            R[b] = jnp.where(hit, vr, R[b])
            Rn[b] = jnp.where(hit, nr, Rn[b])
        return tuple(R), tuple(Rn), tuple(selv), tuple(seln)

    selv0 = tuple(jnp.zeros((1, 128), jnp.float32) for _ in range(_B))
    seln0 = tuple(jnp.zeros((1, 128), jnp.int32) for _ in range(_B))
    _, _, selv, seln = lax.fori_loop(
        0, _TOPN, body, (tuple(R0), tuple(Rn0), selv0, seln0))

    for b in range(_B):
        p_i = seln[b] // 12
        a_i = seln[b] - p_i * 12
        G = jnp.zeros((72, 128), jnp.float32)
        for k in range(8):
            pio = lax.broadcasted_iota(jnp.int32, (1024, 128), 0) + k * 1024
            oneh = (pio == p_i).astype(jnp.float32)
            blk = bbox_ref[b, :, k * 1024:(k + 1) * 1024]
            G = G + lax.dot_general(blk, oneh, (((1,), (0,)), ((), ())),
                                    preferred_element_type=jnp.float32)
        d = jnp.zeros((6, 128), jnp.float32)
        an = [jnp.zeros((1, 128), jnp.float32) for _ in range(6)]
        for a in range(12):
            hit_a = a_i == a
            d = jnp.where(hit_a, G[a * 6:(a + 1) * 6, :], d)
            for jj in range(6):
                an[jj] = jnp.where(hit_a, float(_ANCHORS[a, jj]), an[jj])

        hs = p_i // 256
        ws = (p_i // 8) % 32
        ts = p_i % 8
        sx = (ws * 16).astype(jnp.float32)
        sy = (hs * 16).astype(jnp.float32)
        sz = ts.astype(jnp.float32)
        a0 = an[0] + sx
        a1 = an[1] + sy
        a2 = an[2] + sz
        a3_ = an[3] + sx
        a4 = an[4] + sy
        a5 = an[5] + sz
        w = a3_ - a0 + 1.0
        h = a4 - a1 + 1.0
        l = a5 - a2 + 1.0
        cx = a0 + 0.5 * w
        cy = a1 + 0.5 * h
        ct = a2 + 0.5 * l
        pcx = d[0:1, :] * w + cx
        pcy = d[1:2, :] * h + cy
        pct = d[2:3, :] * l + ct
        pw = jnp.exp(d[3:4, :]) * w
        ph = jnp.exp(d[4:5, :]) * h
        pll = jnp.exp(d[5:6, :]) * l
        Hc = im_ref[b, 0] - 1.0
        Wc = im_ref[b, 1] - 1.0
        Tc = im_ref[b, 2] - 1.0
        x1 = jnp.clip(pcx - 0.5 * pw, 0.0, Wc)
        y1 = jnp.clip(pcy - 0.5 * ph, 0.0, Hc)
        t1 = jnp.clip(pct - 0.5 * pll, 0.0, Tc)
        x2 = jnp.clip(pcx + 0.5 * pw, 0.0, Wc)
        y2 = jnp.clip(pcy + 0.5 * ph, 0.0, Hc)
        t2 = jnp.clip(pct + 0.5 * pll, 0.0, Tc)
        brow = jnp.full((1, 128), float(b), jnp.float32)
        out_ref[b] = jnp.concatenate(
            [brow, x1, y1, t1, x2, y2, t2, selv[b]], axis=0)


def kernel(scores_full, bbox_frame, im_info):
    B = scores_full.shape[0]
    scores = scores_full[:, 12:, :, :, :].reshape(B, 6, 128, 128)
    bbox = bbox_frame.reshape(B, 72, 8192)
    out = pl.pallas_call(
        _proposal_kernel,
        in_specs=[
            pl.BlockSpec((B, 6, 128, 128), lambda: (0, 0, 0, 0)),
            pl.BlockSpec((B, 72, 8192), lambda: (0, 0, 0)),
            pl.BlockSpec(memory_space=pltpu.SMEM),
        ],
        out_specs=pl.BlockSpec((B, 8, 128), lambda: (0, 0, 0)),
        out_shape=jax.ShapeDtypeStruct((B, 8, 128), jnp.float32),
        scratch_shapes=[pltpu.VMEM((6, 128, 128), jnp.float32)
                        for _ in range(B)],
    )(scores, bbox, im_info)
    return out[:, :, :_TOPN].transpose(0, 2, 1)


# speedup vs baseline: 1.9911x; 1.9911x over previous
import numpy as np
import jax
import jax.numpy as jnp
from jax import lax
from jax.experimental import pallas as pl
from jax.experimental.pallas import tpu as pltpu

_TOPN = 100
_B = 4
_K = 8
_BIGN = np.int32(2**30)
_NEG = float("-inf")

_ANCHORS = np.array(
    [[-38., -16., 0., 53., 31., 15.],
     [-84., -40., 0., 99., 55., 15.],
     [-176., -88., 0., 191., 103., 15.],
     [-360., -184., 0., 375., 199., 15.],
     [-24., -24., 0., 39., 39., 15.],
     [-56., -56., 0., 71., 71., 15.],
     [-120., -120., 0., 135., 135., 15.],
     [-248., -248., 0., 263., 263., 15.],
     [-14., -36., 0., 29., 51., 15.],
     [-36., -80., 0., 51., 95., 15.],
     [-80., -168., 0., 95., 183., 15.],
     [-168., -344., 0., 183., 359., 15.]],
    dtype=np.float32)


def _proposal_kernel(scores_ref, bbox_ref, im_ref, out_ref,
                     selv_ref, seln_ref, *scratches):
    riota = lax.broadcasted_iota(jnp.int32, (192, 512), 0)
    m3 = riota * 512 + lax.broadcasted_iota(jnp.int32, (192, 512), 1)
    a3 = m3 // 8192
    n3 = (m3 - a3 * 8192) * 12 + a3
    lane = lax.broadcasted_iota(jnp.int32, (1, 128), 1)

    Cs, Cns = [], []
    for b in range(_B):
        S = scores_ref[b]
        Tv, Tn = [], []
        for t in range(_K):
            cmax = jnp.max(S, axis=0, keepdims=True)
            hit = S == cmax
            minrow = jnp.min(jnp.where(hit, riota, _BIGN),
                             axis=0, keepdims=True)
            knock = hit & (riota == minrow)
            ncol = jnp.min(jnp.where(knock, n3, _BIGN),
                           axis=0, keepdims=True)
            Tv.append(cmax)
            Tn.append(ncol)
            if t < _K - 1:
                S = jnp.where(knock, _NEG, S)
        Cs.append(jnp.concatenate(Tv, axis=0))
        Cns.append(jnp.concatenate(Tn, axis=0))

    def body(i, carry):
        C, selv, seln = [list(x) for x in carry]
        for b in range(_B):
            v = jnp.max(C[b], keepdims=True).reshape(1, 1)
            nsel = jnp.min(jnp.where(C[b] == v, Cns[b], _BIGN),
                           keepdims=True).reshape(1, 1)
            selv[b] = jnp.where(lane == i, v, selv[b])
            seln[b] = jnp.where(lane == i, nsel, seln[b])
            C[b] = jnp.where(Cns[b] == nsel, _NEG, C[b])
        return tuple(C), tuple(selv), tuple(seln)

    selv0 = tuple(jnp.zeros((1, 128), jnp.float32) for _ in range(_B))
    seln0 = tuple(jnp.zeros((1, 128), jnp.int32) for _ in range(_B))
    Cf, selv, seln = lax.fori_loop(
        0, _TOPN, body, (tuple(Cs), selv0, seln0))

    worst = jnp.zeros((1, 1), jnp.int32)
    for b in range(_B):
        depth = jnp.sum((Cf[b] == _NEG).astype(jnp.int32),
                        axis=0, keepdims=True)
        worst = jnp.maximum(worst,
                            jnp.max(depth, keepdims=True).reshape(1, 1))
    for b in range(_B):
        selv_ref[pl.ds(b, 1), :] = selv[b]
        seln_ref[pl.ds(b, 1), :] = seln[b]

    @pl.when(worst[0, 0] >= _K)
    def _fallback():
        for b in range(_B):
            scratches[b][...] = scores_ref[b]

        def fbody(i, carry):
            fv, fn = [list(x) for x in carry]
            for b in range(_B):
                S = scratches[b][...]
                v = jnp.max(S, keepdims=True).reshape(1, 1)
                nsel = jnp.min(jnp.where(S == v, n3, _BIGN),
                               keepdims=True).reshape(1, 1)
                fv[b] = jnp.where(lane == i, v, fv[b])
                fn[b] = jnp.where(lane == i, nsel, fn[b])
                scratches[b][...] = jnp.where(n3 == nsel, _NEG, S)
            return tuple(fv), tuple(fn)

        fv, fn = lax.fori_loop(0, _TOPN, fbody, (selv0, seln0))
        for b in range(_B):
            selv_ref[pl.ds(b, 1), :] = fv[b]
            seln_ref[pl.ds(b, 1), :] = fn[b]

    for b in range(_B):
        seln_b = seln_ref[pl.ds(b, 1), :]
        selv_b = selv_ref[pl.ds(b, 1), :]
        p_i = seln_b // 12
        a_i = seln_b - p_i * 12
        G = jnp.zeros((72, 128), jnp.float32)
        for k in range(8):
            pio = lax.broadcasted_iota(jnp.int32, (1024, 128), 0) + k * 1024
            oneh = (pio == p_i).astype(jnp.float32)
            blk = bbox_ref[b, :, k * 1024:(k + 1) * 1024]
            G = G + lax.dot_general(blk, oneh, (((1,), (0,)), ((), ())),
                                    preferred_element_type=jnp.float32)
        d = jnp.zeros((6, 128), jnp.float32)
        an = [jnp.zeros((1, 128), jnp.float32) for _ in range(6)]
        for a in range(12):
            hit_a = a_i == a
            d = jnp.where(hit_a, G[a * 6:(a + 1) * 6, :], d)
            for jj in range(6):
                an[jj] = jnp.where(hit_a, float(_ANCHORS[a, jj]), an[jj])

        hs = p_i // 256
        ws = (p_i // 8) % 32
        ts = p_i % 8
        sx = (ws * 16).astype(jnp.float32)
        sy = (hs * 16).astype(jnp.float32)
        sz = ts.astype(jnp.float32)
        a0 = an[0] + sx
        a1 = an[1] + sy
        a2 = an[2] + sz
        a3_ = an[3] + sx
        a4 = an[4] + sy
        a5 = an[5] + sz
        w = a3_ - a0 + 1.0
        h = a4 - a1 + 1.0
        l = a5 - a2 + 1.0
        cx = a0 + 0.5 * w
        cy = a1 + 0.5 * h
        ct = a2 + 0.5 * l
        pcx = d[0:1, :] * w + cx
        pcy = d[1:2, :] * h + cy
        pct = d[2:3, :] * l + ct
        pw = jnp.exp(d[3:4, :]) * w
        ph = jnp.exp(d[4:5, :]) * h
        pll = jnp.exp(d[5:6, :]) * l
        Hc = im_ref[b, 0] - 1.0
        Wc = im_ref[b, 1] - 1.0
        Tc = im_ref[b, 2] - 1.0
        x1 = jnp.clip(pcx - 0.5 * pw, 0.0, Wc)
        y1 = jnp.clip(pcy - 0.5 * ph, 0.0, Hc)
        t1 = jnp.clip(pct - 0.5 * pll, 0.0, Tc)
        x2 = jnp.clip(pcx + 0.5 * pw, 0.0, Wc)
        y2 = jnp.clip(pcy + 0.5 * ph, 0.0, Hc)
        t2 = jnp.clip(pct + 0.5 * pll, 0.0, Tc)
        brow = jnp.full((1, 128), float(b), jnp.float32)
        out_ref[b] = jnp.concatenate(
            [brow, x1, y1, t1, x2, y2, t2, selv_b], axis=0)


def kernel(scores_full, bbox_frame, im_info):
    B = scores_full.shape[0]
    scores = scores_full[:, 12:, :, :, :].reshape(B, 192, 512)
    bbox = bbox_frame.reshape(B, 72, 8192)
    out = pl.pallas_call(
        _proposal_kernel,
        in_specs=[
            pl.BlockSpec((B, 192, 512), lambda: (0, 0, 0)),
            pl.BlockSpec((B, 72, 8192), lambda: (0, 0, 0)),
            pl.BlockSpec(memory_space=pltpu.SMEM),
        ],
        out_specs=pl.BlockSpec((B, 8, 128), lambda: (0, 0, 0)),
        out_shape=jax.ShapeDtypeStruct((B, 8, 128), jnp.float32),
        scratch_shapes=[pltpu.VMEM((8, 128), jnp.float32),
                        pltpu.VMEM((8, 128), jnp.int32)]
                       + [pltpu.VMEM((192, 512), jnp.float32)
                          for _ in range(B)],
    )(scores, bbox, im_info)
    return out[:, :, :_TOPN].transpose(0, 2, 1)
